# baseline (speedup 1.0000x reference)
import functools

import jax
import jax.numpy as jnp
from jax import lax
from jax.experimental import pallas as pl
from jax.experimental.pallas import tpu as pltpu
from jax.experimental.pallas import tpu_sc as plsc

_info = plsc.get_sparse_core_info()
_NC, _NS, _L = _info.num_cores, _info.num_subcores, _info.num_lanes
_NW = _NC * _NS


@functools.lru_cache(maxsize=None)
def _build(batch, seq_len, vocab, dim):
    s_per_w = seq_len // _NW
    assert seq_len % _NW == 0 and s_per_w % 8 == 0 and dim % _L == 0

    def body(idx_hbm, pos_hbm, table_hbm, out_hbm, idx_v, pos_v, acc_v,
             s_i, s_pos, s_p0, *sems):
        s_g, s_o = sems[:batch + 1], sems[batch + 1:]
        wid = lax.axis_index("s") * _NC + lax.axis_index("c")
        s0 = wid * s_per_w
        cp_idx = [
            pltpu.async_copy(idx_hbm.at[b, pl.ds(s0, s_per_w)], idx_v.at[b],
                             s_i)
            for b in range(batch)
        ]
        pos_src = pos_hbm.at[pl.ds(s0, s_per_w), :]
        cp_pos = pltpu.async_copy(pos_src, pos_v, s_pos)
        cp_p0 = pltpu.async_copy(pos_src, acc_v.at[0], s_p0)
        cp_g = [None] * (batch + 1)
        cp_p0.wait()
        for cp in cp_idx:
            cp.wait()
        cp_g[0] = pltpu.async_copy(table_hbm.at[idx_v.at[0]], acc_v.at[0],
                                   s_g[0], add=True)
        cp_pos.wait()

        def replicate(b, r0, n):
            def row(i, carry):
                for j in range(dim // _L):
                    sl = pl.ds(j * _L, _L)
                    acc_v[b, r0 + i, sl] = pos_v[r0 + i, sl]
                return carry

            lax.fori_loop(0, n, row, 0)

        half = s_per_w // 2
        for b in range(1, batch - 1):
            replicate(b, 0, s_per_w)
            cp_g[b] = pltpu.async_copy(table_hbm.at[idx_v.at[b]], acc_v.at[b],
                                       s_g[b], add=True)
        lb = batch - 1
        replicate(lb, 0, half)
        cp_g[lb] = pltpu.async_copy(
            table_hbm.at[idx_v.at[lb, pl.ds(0, half)]],
            acc_v.at[lb, pl.ds(0, half), :], s_g[lb], add=True)
        replicate(lb, half, half)
        cp_g[lb + 1] = pltpu.async_copy(
            table_hbm.at[idx_v.at[lb, pl.ds(half, half)]],
            acc_v.at[lb, pl.ds(half, half), :], s_g[lb + 1], add=True)
        cp_o = []
        for b in range(batch - 1):
            cp_g[b].wait()
            cp_o.append(pltpu.async_copy(
                acc_v.at[b], out_hbm.at[b, pl.ds(s0, s_per_w), :], s_o[b]))
        cp_g[lb].wait()
        cp_o.append(pltpu.async_copy(
            acc_v.at[lb, pl.ds(0, half), :],
            out_hbm.at[lb, pl.ds(s0, half), :], s_o[lb]))
        cp_g[lb + 1].wait()
        cp_o.append(pltpu.async_copy(
            acc_v.at[lb, pl.ds(half, half), :],
            out_hbm.at[lb, pl.ds(s0 + half, half), :], s_o[lb + 1]))
        for cp in cp_o:
            cp.wait()

    mesh = plsc.VectorSubcoreMesh(core_axis_name="c", subcore_axis_name="s")
    kern = pl.kernel(
        body,
        mesh=mesh,
        out_type=jax.ShapeDtypeStruct((batch, seq_len, dim), jnp.float32),
        scratch_types=[
            pltpu.VMEM((batch, s_per_w), jnp.int32),
            pltpu.VMEM((s_per_w, dim), jnp.float32),
            pltpu.VMEM((batch, s_per_w, dim), jnp.float32),
            pltpu.SemaphoreType.DMA,
            pltpu.SemaphoreType.DMA,
            pltpu.SemaphoreType.DMA,
        ] + [pltpu.SemaphoreType.DMA] * (2 * batch + 2),
    )

    @jax.jit
    def run(input_ids, token_table, pos_table):
        return kern(input_ids.astype(jnp.int32), pos_table, token_table)

    return run


def kernel(input_ids, token_table, pos_table):
    batch, seq_len = input_ids.shape
    vocab, dim = token_table.shape
    return _build(batch, seq_len, vocab, dim)(input_ids, token_table, pos_table)


# speedup vs baseline: 1.0003x; 1.0003x over previous
import functools

import jax
import jax.numpy as jnp
from jax import lax
from jax.experimental import pallas as pl
from jax.experimental.pallas import tpu as pltpu
from jax.experimental.pallas import tpu_sc as plsc

_info = plsc.get_sparse_core_info()
_NC, _NS, _L = _info.num_cores, _info.num_subcores, _info.num_lanes
_NW = _NC * _NS


@functools.lru_cache(maxsize=None)
def _build(batch, seq_len, vocab, dim):
    s_per_w = seq_len // _NW
    assert seq_len % _NW == 0 and s_per_w % 8 == 0 and dim % _L == 0

    def body(idx_hbm, pos_hbm, table_hbm, out_hbm, idx_v, pos_v, acc_v,
             s_i, s_pos, s_p0, *sems):
        s_g, s_o = sems[:batch], sems[batch:]
        wid = lax.axis_index("s") * _NC + lax.axis_index("c")
        s0 = wid * s_per_w
        cp_idx = [
            pltpu.async_copy(idx_hbm.at[b, pl.ds(s0, s_per_w)], idx_v.at[b],
                             s_i)
            for b in range(batch)
        ]
        pos_src = pos_hbm.at[pl.ds(s0, s_per_w), :]
        cp_pos = pltpu.async_copy(pos_src, pos_v, s_pos)
        cp_p0 = pltpu.async_copy(pos_src, acc_v.at[0], s_p0)
        cp_g = [None] * batch
        cp_p0.wait()
        for cp in cp_idx:
            cp.wait()
        cp_g[0] = pltpu.async_copy(table_hbm.at[idx_v.at[0]], acc_v.at[0],
                                   s_g[0], add=True)
        cp_pos.wait()

        def row(i, carry):
            for j in range(dim // _L):
                sl = pl.ds(j * _L, _L)
                v = pos_v[i, sl]
                for b in range(1, batch):
                    acc_v[b, i, sl] = v
            return carry

        lax.fori_loop(0, s_per_w, row, 0)
        for b in range(1, batch):
            cp_g[b] = pltpu.async_copy(table_hbm.at[idx_v.at[b]], acc_v.at[b],
                                       s_g[b], add=True)
        cp_o = []
        for b in range(batch):
            cp_g[b].wait()
            cp_o.append(pltpu.async_copy(
                acc_v.at[b], out_hbm.at[b, pl.ds(s0, s_per_w), :], s_o[b]))
        for cp in cp_o:
            cp.wait()

    mesh = plsc.VectorSubcoreMesh(core_axis_name="c", subcore_axis_name="s")
    kern = pl.kernel(
        body,
        mesh=mesh,
        out_type=jax.ShapeDtypeStruct((batch, seq_len, dim), jnp.float32),
        scratch_types=[
            pltpu.VMEM((batch, s_per_w), jnp.int32),
            pltpu.VMEM((s_per_w, dim), jnp.float32),
            pltpu.VMEM((batch, s_per_w, dim), jnp.float32),
            pltpu.SemaphoreType.DMA,
            pltpu.SemaphoreType.DMA,
            pltpu.SemaphoreType.DMA,
        ] + [pltpu.SemaphoreType.DMA] * (2 * batch),
    )

    @jax.jit
    def run(input_ids, token_table, pos_table):
        return kern(input_ids.astype(jnp.int32), pos_table, token_table)

    return run


def kernel(input_ids, token_table, pos_table):
    batch, seq_len = input_ids.shape
    vocab, dim = token_table.shape
    return _build(batch, seq_len, vocab, dim)(input_ids, token_table, pos_table)
